# Initial kernel scaffold; baseline (speedup 1.0000x reference)
import jax
import jax.numpy as jnp
from jax.experimental import pallas as pl


def kernel(x, edge_index, W1, b1, W2, b2):
    raise NotImplementedError("write your pallas kernel here")


# speedup vs baseline: 10.1868x; 10.1868x over previous
import functools

import jax
import jax.numpy as jnp
from jax import lax
from jax.experimental import pallas as pl
from jax.experimental.pallas import tpu as pltpu
from jax.experimental.pallas import tpu_sc as plsc

N = 10000
E = 320000
D = 128
NC, NS = 2, 16
NW = NC * NS
CHUNK = 128
EPW = 10240
EPAD = EPW * NW
NCH = EPW // CHUNK
NCHB = 8
NBLK = NCH // NCHB
ROWS = 10240
RPT = ROWS // NS


@functools.lru_cache(maxsize=None)
def _make_sc_pass(with_gather: bool):
  mesh = plsc.VectorSubcoreMesh(
      core_axis_name="c", subcore_axis_name="s", num_cores=NC, num_subcores=NS
  )

  def body(table, srcs, dsts, zrows, acc_out, src_v, dst_v, rows_v, gsem,
           acc_sh):
    cid = lax.axis_index("c")
    sid = lax.axis_index("s")
    wid = cid * NS + sid

    pltpu.sync_copy(zrows, acc_sh.at[pl.ds(sid * RPT, RPT)])
    if not with_gather:
      pltpu.sync_copy(table.at[pl.ds(0, CHUNK)], rows_v)
    plsc.subcore_barrier()

    def block(b, carry):
      pltpu.sync_copy(dsts.at[wid, pl.ds(b * NCHB, NCHB)], dst_v)
      if with_gather:
        pltpu.sync_copy(srcs.at[wid, pl.ds(b * NCHB, NCHB)], src_v)

      def chunk(j, c):
        if with_gather:
          pltpu.async_copy(table.at[src_v.at[j]], rows_v, gsem).wait()
        pltpu.sync_copy(rows_v, acc_sh.at[dst_v.at[j]], add=True)
        return c

      return lax.fori_loop(0, NCHB, chunk, carry)

    lax.fori_loop(0, NBLK, block, 0)
    plsc.subcore_barrier()

    pltpu.sync_copy(acc_sh.at[pl.ds(sid * RPT, RPT)],
                    acc_out.at[cid, pl.ds(sid * RPT, RPT)])

  return pl.kernel(
      body,
      out_type=jax.ShapeDtypeStruct((NC, ROWS, D), jnp.float32),
      mesh=mesh,
      scratch_types=[
          pltpu.VMEM((NCHB, CHUNK), jnp.int32),
          pltpu.VMEM((NCHB, CHUNK), jnp.int32),
          pltpu.VMEM((CHUNK, D), jnp.float32),
          pltpu.SemaphoreType.DMA,
          pltpu.VMEM_SHARED((ROWS, D), jnp.float32),
      ],
  )


def _combine_body(s_ref, g_ref, x_ref, h1_ref):
  s = s_ref[0] + s_ref[1]
  deg = g_ref[0, :, 0:1] + g_ref[1, :, 0:1]
  h1_ref[...] = s / jnp.maximum(deg, 1.0) - x_ref[...] * jnp.minimum(deg, 1.0)


def _final_body(s_ref, g_ref, x_ref, h1_ref, w1x_ref, w1a_ref, w1b_ref,
                b1_ref, w2_ref, b2_ref, out_ref):
  s = s_ref[0] + s_ref[1]
  deg = g_ref[0, :, 0:1] + g_ref[1, :, 0:1]
  x = x_ref[...]
  h1 = h1_ref[...]
  h2 = s / jnp.maximum(deg, 1.0) - h1 * jnp.minimum(deg, 1.0)
  hid = (jnp.dot(x, w1x_ref[...], preferred_element_type=jnp.float32)
         + jnp.dot(h1, w1a_ref[...], preferred_element_type=jnp.float32)
         + jnp.dot(h2, w1b_ref[...], preferred_element_type=jnp.float32)
         + b1_ref[...])
  hid = jnp.maximum(hid, 0.0)
  out_ref[...] = (jnp.dot(hid, w2_ref[...], preferred_element_type=jnp.float32)
                  + b2_ref[...])


_BLK = 1000


def _full(shape):
  return pl.BlockSpec(shape, lambda i: (0,) * len(shape))


_combine = pl.pallas_call(
    _combine_body,
    grid=(N // _BLK,),
    in_specs=[
        pl.BlockSpec((NC, _BLK, D), lambda i: (0, i, 0)),
        pl.BlockSpec((NC, _BLK, D), lambda i: (0, i, 0)),
        pl.BlockSpec((_BLK, D), lambda i: (i, 0)),
    ],
    out_specs=pl.BlockSpec((_BLK, D), lambda i: (i, 0)),
    out_shape=jax.ShapeDtypeStruct((N, D), jnp.float32),
)

_final = pl.pallas_call(
    _final_body,
    grid=(N // _BLK,),
    in_specs=[
        pl.BlockSpec((NC, _BLK, D), lambda i: (0, i, 0)),
        pl.BlockSpec((NC, _BLK, D), lambda i: (0, i, 0)),
        pl.BlockSpec((_BLK, D), lambda i: (i, 0)),
        pl.BlockSpec((_BLK, D), lambda i: (i, 0)),
        _full((D, 16)),
        _full((D, 16)),
        _full((D, 16)),
        _full((1, 16)),
        _full((16, 3)),
        _full((1, 3)),
    ],
    out_specs=pl.BlockSpec((_BLK, 3), lambda i: (i, 0)),
    out_shape=jax.ShapeDtypeStruct((N, 3), jnp.float32),
)


def kernel(x, edge_index, W1, b1, W2, b2):
  x = x.astype(jnp.float32)
  ei = edge_index.astype(jnp.int32)
  npad = EPAD - E
  pad = jnp.arange(npad, dtype=jnp.int32)
  src = jnp.concatenate([ei[0], pad % N]).reshape(NW, NCH, CHUNK)
  dst = jnp.concatenate([ei[1], N + pad % (ROWS - N)]).reshape(NW, NCH, CHUNK)
  zrows = jnp.zeros((RPT, D), jnp.float32)
  ones = jnp.ones((CHUNK, D), jnp.float32)

  degp = _make_sc_pass(False)(ones, src, dst, zrows)
  s1 = _make_sc_pass(True)(x, src, dst, zrows)
  h1 = _combine(s1, degp, x)
  s2 = _make_sc_pass(True)(h1, src, dst, zrows)
  out = _final(s2, degp, x, h1,
               W1[:D], W1[D:2 * D], W1[2 * D:],
               b1.reshape(1, 16), W2, b2.reshape(1, 3))
  return out


# speedup vs baseline: 12.9655x; 1.2728x over previous
import functools

import jax
import jax.numpy as jnp
from jax import lax
from jax.experimental import pallas as pl
from jax.experimental.pallas import tpu as pltpu
from jax.experimental.pallas import tpu_sc as plsc

N = 10000
E = 320000
D = 128
NC, NS = 2, 16
NW = NC * NS
CHUNK = 128
EPW = 10240
EPAD = EPW * NW
NCH = EPW // CHUNK
NCHB = 8
NBLK = NCH // NCHB
ROWS = 10240
RPT = ROWS // NS


@functools.lru_cache(maxsize=None)
def _make_sc_pass(with_gather: bool):
  mesh = plsc.VectorSubcoreMesh(
      core_axis_name="c", subcore_axis_name="s", num_cores=NC, num_subcores=NS
  )

  def body(table, srcs, dsts, zrows, acc_out, src_v, dst_v, rows_a, rows_b,
           sem_a, sem_b, acc_sh):
    cid = lax.axis_index("c")
    sid = lax.axis_index("s")
    wid = cid * NS + sid
    rows = (rows_a, rows_b)
    sems = (sem_a, sem_b)

    pltpu.sync_copy(zrows, acc_sh.at[pl.ds(sid * RPT, RPT)])
    if not with_gather:
      pltpu.sync_copy(table.at[pl.ds(0, CHUNK)], rows_a)
    plsc.subcore_barrier()

    def block(b, carry):
      pltpu.sync_copy(dsts.at[wid, pl.ds(b * NCHB, NCHB)], dst_v)
      if with_gather:
        pltpu.sync_copy(srcs.at[wid, pl.ds(b * NCHB, NCHB)], src_v)
        descs = [None] * NCHB
        descs[0] = pltpu.async_copy(table.at[src_v.at[0]], rows[0], sems[0])
        for j in range(NCHB):
          if j + 1 < NCHB:
            descs[j + 1] = pltpu.async_copy(
                table.at[src_v.at[j + 1]], rows[(j + 1) % 2], sems[(j + 1) % 2])
          descs[j].wait()
          pltpu.sync_copy(rows[j % 2], acc_sh.at[dst_v.at[j]], add=True)
      else:
        for j in range(NCHB):
          pltpu.sync_copy(rows_a, acc_sh.at[dst_v.at[j]], add=True)
      return carry

    lax.fori_loop(0, NBLK, block, 0)
    plsc.subcore_barrier()

    pltpu.sync_copy(acc_sh.at[pl.ds(sid * RPT, RPT)],
                    acc_out.at[cid, pl.ds(sid * RPT, RPT)])

  return pl.kernel(
      body,
      out_type=jax.ShapeDtypeStruct((NC, ROWS, D), jnp.float32),
      mesh=mesh,
      scratch_types=[
          pltpu.VMEM((NCHB, CHUNK), jnp.int32),
          pltpu.VMEM((NCHB, CHUNK), jnp.int32),
          pltpu.VMEM((CHUNK, D), jnp.float32),
          pltpu.VMEM((CHUNK, D), jnp.float32),
          pltpu.SemaphoreType.DMA,
          pltpu.SemaphoreType.DMA,
          pltpu.VMEM_SHARED((ROWS, D), jnp.float32),
      ],
  )


def _combine_body(s_ref, g_ref, x_ref, h1_ref):
  s = s_ref[0] + s_ref[1]
  deg = g_ref[0, :, 0:1] + g_ref[1, :, 0:1]
  h1_ref[...] = s / jnp.maximum(deg, 1.0) - x_ref[...] * jnp.minimum(deg, 1.0)


def _final_body(s_ref, g_ref, x_ref, h1_ref, w1x_ref, w1a_ref, w1b_ref,
                b1_ref, w2_ref, b2_ref, out_ref):
  s = s_ref[0] + s_ref[1]
  deg = g_ref[0, :, 0:1] + g_ref[1, :, 0:1]
  x = x_ref[...]
  h1 = h1_ref[...]
  h2 = s / jnp.maximum(deg, 1.0) - h1 * jnp.minimum(deg, 1.0)
  hid = (jnp.dot(x, w1x_ref[...], preferred_element_type=jnp.float32)
         + jnp.dot(h1, w1a_ref[...], preferred_element_type=jnp.float32)
         + jnp.dot(h2, w1b_ref[...], preferred_element_type=jnp.float32)
         + b1_ref[...])
  hid = jnp.maximum(hid, 0.0)
  out_ref[...] = (jnp.dot(hid, w2_ref[...], preferred_element_type=jnp.float32)
                  + b2_ref[...])


_BLK = 1000


def _full(shape):
  return pl.BlockSpec(shape, lambda i: (0,) * len(shape))


_combine = pl.pallas_call(
    _combine_body,
    grid=(N // _BLK,),
    in_specs=[
        pl.BlockSpec((NC, _BLK, D), lambda i: (0, i, 0)),
        pl.BlockSpec((NC, _BLK, D), lambda i: (0, i, 0)),
        pl.BlockSpec((_BLK, D), lambda i: (i, 0)),
    ],
    out_specs=pl.BlockSpec((_BLK, D), lambda i: (i, 0)),
    out_shape=jax.ShapeDtypeStruct((N, D), jnp.float32),
)

_final = pl.pallas_call(
    _final_body,
    grid=(N // _BLK,),
    in_specs=[
        pl.BlockSpec((NC, _BLK, D), lambda i: (0, i, 0)),
        pl.BlockSpec((NC, _BLK, D), lambda i: (0, i, 0)),
        pl.BlockSpec((_BLK, D), lambda i: (i, 0)),
        pl.BlockSpec((_BLK, D), lambda i: (i, 0)),
        _full((D, 16)),
        _full((D, 16)),
        _full((D, 16)),
        _full((1, 16)),
        _full((16, 3)),
        _full((1, 3)),
    ],
    out_specs=pl.BlockSpec((_BLK, 3), lambda i: (i, 0)),
    out_shape=jax.ShapeDtypeStruct((N, 3), jnp.float32),
)


def kernel(x, edge_index, W1, b1, W2, b2):
  x = x.astype(jnp.float32)
  ei = edge_index.astype(jnp.int32)
  npad = EPAD - E
  pad = jnp.arange(npad, dtype=jnp.int32)
  src = jnp.concatenate([ei[0], pad % N]).reshape(NW, NCH, CHUNK)
  dst = jnp.concatenate([ei[1], N + pad % (ROWS - N)]).reshape(NW, NCH, CHUNK)
  zrows = jnp.zeros((RPT, D), jnp.float32)
  ones = jnp.ones((CHUNK, D), jnp.float32)

  degp = _make_sc_pass(False)(ones, src, dst, zrows)
  s1 = _make_sc_pass(True)(x, src, dst, zrows)
  h1 = _combine(s1, degp, x)
  s2 = _make_sc_pass(True)(h1, src, dst, zrows)
  out = _final(s2, degp, x, h1,
               W1[:D], W1[D:2 * D], W1[2 * D:],
               b1.reshape(1, 16), W2, b2.reshape(1, 3))
  return out


# speedup vs baseline: 13.8692x; 1.0697x over previous
import functools

import jax
import jax.numpy as jnp
from jax import lax
from jax.experimental import pallas as pl
from jax.experimental.pallas import tpu as pltpu
from jax.experimental.pallas import tpu_sc as plsc

N = 10000
E = 320000
D = 128
NC, NS = 2, 16
NW = NC * NS
CHUNK = 128
EPW = 10240
EPAD = EPW * NW
NCH = EPW // CHUNK
NCHB = 8
NBLK = NCH // NCHB
ROWS = 10240
RPT = ROWS // NS


@functools.lru_cache(maxsize=None)
def _make_sc_pass(with_gather: bool):
  mesh = plsc.VectorSubcoreMesh(
      core_axis_name="c", subcore_axis_name="s", num_cores=NC, num_subcores=NS
  )

  def body(table, srcs, dsts, zrows, acc_out, src_v, dst_v, rows_a, rows_b,
           sem_a, sem_b, isem, acc_sh):
    cid = lax.axis_index("c")
    sid = lax.axis_index("s")
    wid = cid * NS + sid
    rows = (rows_a, rows_b)
    sems = (sem_a, sem_b)

    pltpu.sync_copy(zrows, acc_sh.at[pl.ds(sid * RPT, RPT)])
    if not with_gather:
      pltpu.sync_copy(table.at[pl.ds(0, CHUNK)], rows_a)
    pltpu.sync_copy(dsts.at[wid, pl.ds(0, NCHB)], dst_v.at[pl.ds(0, NCHB)])
    if with_gather:
      pltpu.sync_copy(srcs.at[wid, pl.ds(0, NCHB)], src_v.at[pl.ds(0, NCHB)])
    plsc.subcore_barrier()

    def block(b, carry):
      half = (b % 2) * NCHB
      nxt = ((b + 1) % 2) * NCHB

      @pl.when(b + 1 < NBLK)
      def _prefetch():
        pltpu.async_copy(dsts.at[wid, pl.ds((b + 1) * NCHB, NCHB)],
                         dst_v.at[pl.ds(nxt, NCHB)], isem)
        if with_gather:
          pltpu.async_copy(srcs.at[wid, pl.ds((b + 1) * NCHB, NCHB)],
                           src_v.at[pl.ds(nxt, NCHB)], isem)

      if with_gather:
        descs = [None] * NCHB
        descs[0] = pltpu.async_copy(
            table.at[src_v.at[half + 0]], rows[0], sems[0])
        for j in range(NCHB):
          if j + 1 < NCHB:
            descs[j + 1] = pltpu.async_copy(
                table.at[src_v.at[half + j + 1]],
                rows[(j + 1) % 2], sems[(j + 1) % 2])
          descs[j].wait()
          pltpu.sync_copy(rows[j % 2], acc_sh.at[dst_v.at[half + j]], add=True)
      else:
        for j in range(NCHB):
          pltpu.sync_copy(rows_a, acc_sh.at[dst_v.at[half + j]], add=True)

      @pl.when(b + 1 < NBLK)
      def _drain():
        pltpu.make_async_copy(dsts.at[wid, pl.ds((b + 1) * NCHB, NCHB)],
                              dst_v.at[pl.ds(nxt, NCHB)], isem).wait()
        if with_gather:
          pltpu.make_async_copy(srcs.at[wid, pl.ds((b + 1) * NCHB, NCHB)],
                                src_v.at[pl.ds(nxt, NCHB)], isem).wait()

      return carry

    lax.fori_loop(0, NBLK, block, 0)
    plsc.subcore_barrier()

    pltpu.sync_copy(acc_sh.at[pl.ds(sid * RPT, RPT)],
                    acc_out.at[cid, pl.ds(sid * RPT, RPT)])

  return pl.kernel(
      body,
      out_type=jax.ShapeDtypeStruct((NC, ROWS, D), jnp.float32),
      mesh=mesh,
      scratch_types=[
          pltpu.VMEM((2 * NCHB, CHUNK), jnp.int32),
          pltpu.VMEM((2 * NCHB, CHUNK), jnp.int32),
          pltpu.VMEM((CHUNK, D), jnp.float32),
          pltpu.VMEM((CHUNK, D), jnp.float32),
          pltpu.SemaphoreType.DMA,
          pltpu.SemaphoreType.DMA,
          pltpu.SemaphoreType.DMA,
          pltpu.VMEM_SHARED((ROWS, D), jnp.float32),
      ],
  )


def _combine_body(s_ref, g_ref, x_ref, h1_ref):
  s = s_ref[0] + s_ref[1]
  deg = g_ref[0, :, 0:1] + g_ref[1, :, 0:1]
  h1_ref[...] = s / jnp.maximum(deg, 1.0) - x_ref[...] * jnp.minimum(deg, 1.0)


def _final_body(s_ref, g_ref, x_ref, h1_ref, w1x_ref, w1a_ref, w1b_ref,
                b1_ref, w2_ref, b2_ref, out_ref):
  s = s_ref[0] + s_ref[1]
  deg = g_ref[0, :, 0:1] + g_ref[1, :, 0:1]
  x = x_ref[...]
  h1 = h1_ref[...]
  h2 = s / jnp.maximum(deg, 1.0) - h1 * jnp.minimum(deg, 1.0)
  hid = (jnp.dot(x, w1x_ref[...], preferred_element_type=jnp.float32)
         + jnp.dot(h1, w1a_ref[...], preferred_element_type=jnp.float32)
         + jnp.dot(h2, w1b_ref[...], preferred_element_type=jnp.float32)
         + b1_ref[...])
  hid = jnp.maximum(hid, 0.0)
  out_ref[...] = (jnp.dot(hid, w2_ref[...], preferred_element_type=jnp.float32)
                  + b2_ref[...])


_BLK = 1000


def _full(shape):
  return pl.BlockSpec(shape, lambda i: (0,) * len(shape))


_combine = pl.pallas_call(
    _combine_body,
    grid=(N // _BLK,),
    in_specs=[
        pl.BlockSpec((NC, _BLK, D), lambda i: (0, i, 0)),
        pl.BlockSpec((NC, _BLK, D), lambda i: (0, i, 0)),
        pl.BlockSpec((_BLK, D), lambda i: (i, 0)),
    ],
    out_specs=pl.BlockSpec((_BLK, D), lambda i: (i, 0)),
    out_shape=jax.ShapeDtypeStruct((N, D), jnp.float32),
)

_final = pl.pallas_call(
    _final_body,
    grid=(N // _BLK,),
    in_specs=[
        pl.BlockSpec((NC, _BLK, D), lambda i: (0, i, 0)),
        pl.BlockSpec((NC, _BLK, D), lambda i: (0, i, 0)),
        pl.BlockSpec((_BLK, D), lambda i: (i, 0)),
        pl.BlockSpec((_BLK, D), lambda i: (i, 0)),
        _full((D, 16)),
        _full((D, 16)),
        _full((D, 16)),
        _full((1, 16)),
        _full((16, 3)),
        _full((1, 3)),
    ],
    out_specs=pl.BlockSpec((_BLK, 3), lambda i: (i, 0)),
    out_shape=jax.ShapeDtypeStruct((N, 3), jnp.float32),
)


def kernel(x, edge_index, W1, b1, W2, b2):
  x = x.astype(jnp.float32)
  ei = edge_index.astype(jnp.int32)
  npad = EPAD - E
  pad = jnp.arange(npad, dtype=jnp.int32)
  src = jnp.concatenate([ei[0], pad % N]).reshape(NW, NCH, CHUNK)
  dst = jnp.concatenate([ei[1], N + pad % (ROWS - N)]).reshape(NW, NCH, CHUNK)
  zrows = jnp.zeros((RPT, D), jnp.float32)
  ones = jnp.ones((CHUNK, D), jnp.float32)

  degp = _make_sc_pass(False)(ones, src, dst, zrows)
  s1 = _make_sc_pass(True)(x, src, dst, zrows)
  h1 = _combine(s1, degp, x)
  s2 = _make_sc_pass(True)(h1, src, dst, zrows)
  out = _final(s2, degp, x, h1,
               W1[:D], W1[D:2 * D], W1[2 * D:],
               b1.reshape(1, 16), W2, b2.reshape(1, 3))
  return out


# speedup vs baseline: 14.0873x; 1.0157x over previous
import functools

import jax
import jax.numpy as jnp
from jax import lax
from jax.experimental import pallas as pl
from jax.experimental.pallas import tpu as pltpu
from jax.experimental.pallas import tpu_sc as plsc

N = 10000
E = 320000
D = 128
NC, NS = 2, 16
NW = NC * NS
CHUNK = 128
EPW = 10240
EPAD = EPW * NW
NCH = EPW // CHUNK
NCHB = 8
NBLK = NCH // NCHB
ROWS = 10240
RPT = ROWS // NS


def _scatter_loop(with_gather, table, srcs, dsts, src_v, dst_v, rows, sems,
                  isem, acc_sh, wid):
  def run():
    pltpu.sync_copy(dsts.at[wid, pl.ds(0, NCHB)], dst_v.at[pl.ds(0, NCHB)])
    if with_gather:
      pltpu.sync_copy(srcs.at[wid, pl.ds(0, NCHB)], src_v.at[pl.ds(0, NCHB)])

    def block(b, carry):
      half = (b % 2) * NCHB
      nxt = ((b + 1) % 2) * NCHB

      @pl.when(b + 1 < NBLK)
      def _prefetch():
        pltpu.async_copy(dsts.at[wid, pl.ds((b + 1) * NCHB, NCHB)],
                         dst_v.at[pl.ds(nxt, NCHB)], isem)
        if with_gather:
          pltpu.async_copy(srcs.at[wid, pl.ds((b + 1) * NCHB, NCHB)],
                           src_v.at[pl.ds(nxt, NCHB)], isem)

      if with_gather:
        descs = [None] * NCHB
        descs[0] = pltpu.async_copy(
            table.at[src_v.at[half + 0]], rows[0], sems[0])
        for j in range(NCHB):
          if j + 1 < NCHB:
            descs[j + 1] = pltpu.async_copy(
                table.at[src_v.at[half + j + 1]],
                rows[(j + 1) % 2], sems[(j + 1) % 2])
          descs[j].wait()
          pltpu.sync_copy(rows[j % 2], acc_sh.at[dst_v.at[half + j]], add=True)
      else:
        for j in range(NCHB):
          pltpu.sync_copy(rows[0], acc_sh.at[dst_v.at[half + j]], add=True)

      @pl.when(b + 1 < NBLK)
      def _drain():
        pltpu.make_async_copy(dsts.at[wid, pl.ds((b + 1) * NCHB, NCHB)],
                              dst_v.at[pl.ds(nxt, NCHB)], isem).wait()
        if with_gather:
          pltpu.make_async_copy(srcs.at[wid, pl.ds((b + 1) * NCHB, NCHB)],
                                src_v.at[pl.ds(nxt, NCHB)], isem).wait()

      return carry

    lax.fori_loop(0, NBLK, block, 0)

  return run


_SC_SCRATCH = lambda: [
    pltpu.VMEM((2 * NCHB, CHUNK), jnp.int32),
    pltpu.VMEM((2 * NCHB, CHUNK), jnp.int32),
    pltpu.VMEM((CHUNK, D), jnp.float32),
    pltpu.VMEM((CHUNK, D), jnp.float32),
    pltpu.SemaphoreType.DMA,
    pltpu.SemaphoreType.DMA,
    pltpu.SemaphoreType.DMA,
    pltpu.VMEM_SHARED((ROWS, D), jnp.float32),
]


def _mesh():
  return plsc.VectorSubcoreMesh(
      core_axis_name="c", subcore_axis_name="s", num_cores=NC, num_subcores=NS
  )


@functools.lru_cache(maxsize=None)
def _make_deg_s1():
  def body(table, ones, srcs, dsts, zrows, s1_out, deg_out, src_v, dst_v,
           rows_a, rows_b, sem_a, sem_b, isem, acc_sh):
    cid = lax.axis_index("c")
    sid = lax.axis_index("s")
    wid = cid * NS + sid
    rows = (rows_a, rows_b)
    sems = (sem_a, sem_b)
    my = pl.ds(sid * RPT, RPT)

    pltpu.sync_copy(zrows, acc_sh.at[my])
    pltpu.sync_copy(ones, rows_a)
    plsc.subcore_barrier()
    _scatter_loop(False, table, srcs, dsts, src_v, dst_v, rows, sems, isem,
                  acc_sh, wid)()
    plsc.subcore_barrier()
    pltpu.sync_copy(acc_sh.at[my], deg_out.at[cid, my])
    pltpu.sync_copy(zrows, acc_sh.at[my])
    plsc.subcore_barrier()
    _scatter_loop(True, table, srcs, dsts, src_v, dst_v, rows, sems, isem,
                  acc_sh, wid)()
    plsc.subcore_barrier()
    pltpu.sync_copy(acc_sh.at[my], s1_out.at[cid, my])

  return pl.kernel(
      body,
      out_type=(jax.ShapeDtypeStruct((NC, ROWS, D), jnp.float32),
                jax.ShapeDtypeStruct((NC, ROWS, D), jnp.float32)),
      mesh=_mesh(),
      scratch_types=_SC_SCRATCH(),
  )


@functools.lru_cache(maxsize=None)
def _make_s_pass():
  def body(table, srcs, dsts, zrows, acc_out, src_v, dst_v, rows_a, rows_b,
           sem_a, sem_b, isem, acc_sh):
    cid = lax.axis_index("c")
    sid = lax.axis_index("s")
    wid = cid * NS + sid
    my = pl.ds(sid * RPT, RPT)

    pltpu.sync_copy(zrows, acc_sh.at[my])
    plsc.subcore_barrier()
    _scatter_loop(True, table, srcs, dsts, src_v, dst_v, (rows_a, rows_b),
                  (sem_a, sem_b), isem, acc_sh, wid)()
    plsc.subcore_barrier()
    pltpu.sync_copy(acc_sh.at[my], acc_out.at[cid, my])

  return pl.kernel(
      body,
      out_type=jax.ShapeDtypeStruct((NC, ROWS, D), jnp.float32),
      mesh=_mesh(),
      scratch_types=_SC_SCRATCH(),
  )


def _combine_body(s_ref, g_ref, x_ref, h1_ref):
  s = s_ref[0] + s_ref[1]
  deg = g_ref[0, :, 0:1] + g_ref[1, :, 0:1]
  h1_ref[...] = s / jnp.maximum(deg, 1.0) - x_ref[...] * jnp.minimum(deg, 1.0)


def _final_body(s_ref, g_ref, x_ref, h1_ref, w1x_ref, w1a_ref, w1b_ref,
                b1_ref, w2_ref, b2_ref, out_ref):
  s = s_ref[0] + s_ref[1]
  deg = g_ref[0, :, 0:1] + g_ref[1, :, 0:1]
  x = x_ref[...]
  h1 = h1_ref[...]
  h2 = s / jnp.maximum(deg, 1.0) - h1 * jnp.minimum(deg, 1.0)
  hid = (jnp.dot(x, w1x_ref[...], preferred_element_type=jnp.float32)
         + jnp.dot(h1, w1a_ref[...], preferred_element_type=jnp.float32)
         + jnp.dot(h2, w1b_ref[...], preferred_element_type=jnp.float32)
         + b1_ref[...])
  hid = jnp.maximum(hid, 0.0)
  out_ref[...] = (jnp.dot(hid, w2_ref[...], preferred_element_type=jnp.float32)
                  + b2_ref[...])


_BLK = 1000


def _full(shape):
  return pl.BlockSpec(shape, lambda i: (0,) * len(shape))


_combine = pl.pallas_call(
    _combine_body,
    grid=(N // _BLK,),
    in_specs=[
        pl.BlockSpec((NC, _BLK, D), lambda i: (0, i, 0)),
        pl.BlockSpec((NC, _BLK, D), lambda i: (0, i, 0)),
        pl.BlockSpec((_BLK, D), lambda i: (i, 0)),
    ],
    out_specs=pl.BlockSpec((_BLK, D), lambda i: (i, 0)),
    out_shape=jax.ShapeDtypeStruct((N, D), jnp.float32),
)

_final = pl.pallas_call(
    _final_body,
    grid=(N // _BLK,),
    in_specs=[
        pl.BlockSpec((NC, _BLK, D), lambda i: (0, i, 0)),
        pl.BlockSpec((NC, _BLK, D), lambda i: (0, i, 0)),
        pl.BlockSpec((_BLK, D), lambda i: (i, 0)),
        pl.BlockSpec((_BLK, D), lambda i: (i, 0)),
        _full((D, 16)),
        _full((D, 16)),
        _full((D, 16)),
        _full((1, 16)),
        _full((16, 3)),
        _full((1, 3)),
    ],
    out_specs=pl.BlockSpec((_BLK, 3), lambda i: (i, 0)),
    out_shape=jax.ShapeDtypeStruct((N, 3), jnp.float32),
)


def kernel(x, edge_index, W1, b1, W2, b2):
  x = x.astype(jnp.float32)
  ei = edge_index.astype(jnp.int32)
  npad = EPAD - E
  pad = jnp.arange(npad, dtype=jnp.int32)
  src = jnp.concatenate([ei[0], pad % N]).reshape(NW, NCH, CHUNK)
  dst = jnp.concatenate([ei[1], N + pad % (ROWS - N)]).reshape(NW, NCH, CHUNK)
  zrows = jnp.zeros((RPT, D), jnp.float32)
  ones = jnp.ones((CHUNK, D), jnp.float32)

  s1, degp = _make_deg_s1()(x, ones, src, dst, zrows)
  h1 = _combine(s1, degp, x)
  s2 = _make_s_pass()(h1, src, dst, zrows)
  out = _final(s2, degp, x, h1,
               W1[:D], W1[D:2 * D], W1[2 * D:],
               b1.reshape(1, 16), W2, b2.reshape(1, 3))
  return out


# speedup vs baseline: 15.0738x; 1.0700x over previous
import functools

import jax
import jax.numpy as jnp
from jax import lax
from jax.experimental import pallas as pl
from jax.experimental.pallas import tpu as pltpu
from jax.experimental.pallas import tpu_sc as plsc

N = 10000
E = 320000
D = 128
NC, NS = 2, 16
NW = NC * NS
CHUNK = 128
EPW = 10240
EPAD = EPW * NW
NCH = EPW // CHUNK
NCHB = 8
NBLK = NCH // NCHB
ROWS = 10240
RPT = ROWS // NS


def _scatter_loop(with_gather, table, srcs, dsts, src_v, dst_v, rows, sems,
                  ssems, isem, acc_sh, wid):
  def _wait_gather(p):
    pltpu.make_async_copy(table.at[pl.ds(0, CHUNK)], rows[p], sems[p]).wait()

  def _wait_scatter(p):
    pltpu.make_async_copy(table.at[pl.ds(0, CHUNK)], rows[p], ssems[p]).wait()

  def run():
    pltpu.sync_copy(dsts.at[wid, pl.ds(0, NCHB)], dst_v.at[pl.ds(0, NCHB)])
    if with_gather:
      pltpu.sync_copy(srcs.at[wid, pl.ds(0, NCHB)], src_v.at[pl.ds(0, NCHB)])
      pltpu.async_copy(table.at[src_v.at[0]], rows[0], sems[0])

    def block(b, carry):
      half = (b % 2) * NCHB
      nxt = ((b + 1) % 2) * NCHB

      if not with_gather:
        @pl.when(b > 0)
        def _drain_prev():
          for _ in range(NCHB):
            _wait_scatter(0)
      else:
        @pl.when(b > 0)
        def _drain_prev_g():
          _wait_scatter(1)

      @pl.when(b + 1 < NBLK)
      def _prefetch():
        pltpu.async_copy(dsts.at[wid, pl.ds((b + 1) * NCHB, NCHB)],
                         dst_v.at[pl.ds(nxt, NCHB)], isem)
        if with_gather:
          pltpu.async_copy(srcs.at[wid, pl.ds((b + 1) * NCHB, NCHB)],
                           src_v.at[pl.ds(nxt, NCHB)], isem)

      if with_gather:
        for j in range(NCHB):
          p = j % 2
          q = 1 - p
          if j > 0:
            _wait_scatter(q)
          if j == NCHB - 1:
            @pl.when(b + 1 < NBLK)
            def _drain_i():
              pltpu.make_async_copy(
                  dsts.at[wid, pl.ds((b + 1) * NCHB, NCHB)],
                  dst_v.at[pl.ds(nxt, NCHB)], isem).wait()
              pltpu.make_async_copy(
                  srcs.at[wid, pl.ds((b + 1) * NCHB, NCHB)],
                  src_v.at[pl.ds(nxt, NCHB)], isem).wait()
            pltpu.async_copy(table.at[src_v.at[nxt]], rows[q], sems[q])
          else:
            pltpu.async_copy(table.at[src_v.at[half + j + 1]], rows[q],
                             sems[q])
          _wait_gather(p)
          pltpu.async_copy(rows[p], acc_sh.at[dst_v.at[half + j]], ssems[p],
                           add=True)
      else:
        for j in range(NCHB):
          pltpu.async_copy(rows[0], acc_sh.at[dst_v.at[half + j]], ssems[0],
                           add=True)

        @pl.when(b + 1 < NBLK)
        def _drain_i2():
          pltpu.make_async_copy(dsts.at[wid, pl.ds((b + 1) * NCHB, NCHB)],
                                dst_v.at[pl.ds(nxt, NCHB)], isem).wait()

      return carry

    lax.fori_loop(0, NBLK, block, 0)
    if with_gather:
      _wait_gather(0)
      _wait_scatter(1)
    else:
      for _ in range(NCHB):
        _wait_scatter(0)

  return run


_SC_SCRATCH = lambda: [
    pltpu.VMEM((2 * NCHB, CHUNK), jnp.int32),
    pltpu.VMEM((2 * NCHB, CHUNK), jnp.int32),
    pltpu.VMEM((CHUNK, D), jnp.float32),
    pltpu.VMEM((CHUNK, D), jnp.float32),
    pltpu.SemaphoreType.DMA,
    pltpu.SemaphoreType.DMA,
    pltpu.SemaphoreType.DMA,
    pltpu.SemaphoreType.DMA,
    pltpu.SemaphoreType.DMA,
    pltpu.VMEM_SHARED((ROWS, D), jnp.float32),
]


def _mesh():
  return plsc.VectorSubcoreMesh(
      core_axis_name="c", subcore_axis_name="s", num_cores=NC, num_subcores=NS
  )


@functools.lru_cache(maxsize=None)
def _make_deg_s1():
  def body(table, ones, srcs, dsts, zrows, s1_out, deg_out, src_v, dst_v,
           rows_a, rows_b, sem_a, sem_b, ssem_a, ssem_b, isem, acc_sh):
    cid = lax.axis_index("c")
    sid = lax.axis_index("s")
    wid = cid * NS + sid
    rows = (rows_a, rows_b)
    sems = (sem_a, sem_b)
    ssems = (ssem_a, ssem_b)
    my = pl.ds(sid * RPT, RPT)

    pltpu.sync_copy(zrows, acc_sh.at[my])
    pltpu.sync_copy(ones, rows_a)
    plsc.subcore_barrier()
    _scatter_loop(False, table, srcs, dsts, src_v, dst_v, rows, sems, ssems,
                  isem, acc_sh, wid)()
    plsc.subcore_barrier()
    pltpu.sync_copy(acc_sh.at[my], deg_out.at[cid, my])
    pltpu.sync_copy(zrows, acc_sh.at[my])
    plsc.subcore_barrier()
    _scatter_loop(True, table, srcs, dsts, src_v, dst_v, rows, sems, ssems,
                  isem, acc_sh, wid)()
    plsc.subcore_barrier()
    pltpu.sync_copy(acc_sh.at[my], s1_out.at[cid, my])

  return pl.kernel(
      body,
      out_type=(jax.ShapeDtypeStruct((NC, ROWS, D), jnp.float32),
                jax.ShapeDtypeStruct((NC, ROWS, D), jnp.float32)),
      mesh=_mesh(),
      scratch_types=_SC_SCRATCH(),
  )


@functools.lru_cache(maxsize=None)
def _make_s_pass():
  def body(table, srcs, dsts, zrows, acc_out, src_v, dst_v, rows_a, rows_b,
           sem_a, sem_b, ssem_a, ssem_b, isem, acc_sh):
    cid = lax.axis_index("c")
    sid = lax.axis_index("s")
    wid = cid * NS + sid
    my = pl.ds(sid * RPT, RPT)

    pltpu.sync_copy(zrows, acc_sh.at[my])
    plsc.subcore_barrier()
    _scatter_loop(True, table, srcs, dsts, src_v, dst_v, (rows_a, rows_b),
                  (sem_a, sem_b), (ssem_a, ssem_b), isem, acc_sh, wid)()
    plsc.subcore_barrier()
    pltpu.sync_copy(acc_sh.at[my], acc_out.at[cid, my])

  return pl.kernel(
      body,
      out_type=jax.ShapeDtypeStruct((NC, ROWS, D), jnp.float32),
      mesh=_mesh(),
      scratch_types=_SC_SCRATCH(),
  )


def _combine_body(s_ref, g_ref, x_ref, h1_ref):
  s = s_ref[0] + s_ref[1]
  deg = g_ref[0, :, 0:1] + g_ref[1, :, 0:1]
  h1_ref[...] = s / jnp.maximum(deg, 1.0) - x_ref[...] * jnp.minimum(deg, 1.0)


def _final_body(s_ref, g_ref, x_ref, h1_ref, w1x_ref, w1a_ref, w1b_ref,
                b1_ref, w2_ref, b2_ref, out_ref):
  s = s_ref[0] + s_ref[1]
  deg = g_ref[0, :, 0:1] + g_ref[1, :, 0:1]
  x = x_ref[...]
  h1 = h1_ref[...]
  h2 = s / jnp.maximum(deg, 1.0) - h1 * jnp.minimum(deg, 1.0)
  hid = (jnp.dot(x, w1x_ref[...], preferred_element_type=jnp.float32)
         + jnp.dot(h1, w1a_ref[...], preferred_element_type=jnp.float32)
         + jnp.dot(h2, w1b_ref[...], preferred_element_type=jnp.float32)
         + b1_ref[...])
  hid = jnp.maximum(hid, 0.0)
  out_ref[...] = (jnp.dot(hid, w2_ref[...], preferred_element_type=jnp.float32)
                  + b2_ref[...])


_BLK = 1000


def _full(shape):
  return pl.BlockSpec(shape, lambda i: (0,) * len(shape))


_combine = pl.pallas_call(
    _combine_body,
    grid=(N // _BLK,),
    in_specs=[
        pl.BlockSpec((NC, _BLK, D), lambda i: (0, i, 0)),
        pl.BlockSpec((NC, _BLK, D), lambda i: (0, i, 0)),
        pl.BlockSpec((_BLK, D), lambda i: (i, 0)),
    ],
    out_specs=pl.BlockSpec((_BLK, D), lambda i: (i, 0)),
    out_shape=jax.ShapeDtypeStruct((N, D), jnp.float32),
)

_final = pl.pallas_call(
    _final_body,
    grid=(N // _BLK,),
    in_specs=[
        pl.BlockSpec((NC, _BLK, D), lambda i: (0, i, 0)),
        pl.BlockSpec((NC, _BLK, D), lambda i: (0, i, 0)),
        pl.BlockSpec((_BLK, D), lambda i: (i, 0)),
        pl.BlockSpec((_BLK, D), lambda i: (i, 0)),
        _full((D, 16)),
        _full((D, 16)),
        _full((D, 16)),
        _full((1, 16)),
        _full((16, 3)),
        _full((1, 3)),
    ],
    out_specs=pl.BlockSpec((_BLK, 3), lambda i: (i, 0)),
    out_shape=jax.ShapeDtypeStruct((N, 3), jnp.float32),
)


def kernel(x, edge_index, W1, b1, W2, b2):
  x = x.astype(jnp.float32)
  ei = edge_index.astype(jnp.int32)
  npad = EPAD - E
  pad = jnp.arange(npad, dtype=jnp.int32)
  src = jnp.concatenate([ei[0], pad % N]).reshape(NW, NCH, CHUNK)
  dst = jnp.concatenate([ei[1], N + pad % (ROWS - N)]).reshape(NW, NCH, CHUNK)
  zrows = jnp.zeros((RPT, D), jnp.float32)
  ones = jnp.ones((CHUNK, D), jnp.float32)

  s1, degp = _make_deg_s1()(x, ones, src, dst, zrows)
  h1 = _combine(s1, degp, x)
  s2 = _make_s_pass()(h1, src, dst, zrows)
  out = _final(s2, degp, x, h1,
               W1[:D], W1[D:2 * D], W1[2 * D:],
               b1.reshape(1, 16), W2, b2.reshape(1, 3))
  return out
